# baseline (speedup 1.0000x reference)
import jax
import jax.numpy as jnp
from jax.experimental import pallas as pl
from jax.experimental.pallas import tpu as pltpu

D_MODEL = 768
NUM_EXPERTS = 64
N_TOKENS = 32768
BT = 2048


def _router_body(x_ref, w_ref, b_ref, logits_ref, probs_ref):
    x = x_ref[...]
    w = w_ref[...]
    logits = jax.lax.dot_general(
        x, w, (((1,), (1,)), ((), ())), preferred_element_type=jnp.float32
    )
    logits = logits + b_ref[...]
    logits_ref[...] = logits
    m = jnp.max(logits, axis=-1, keepdims=True)
    e = jnp.exp(logits - m)
    probs_ref[...] = e / jnp.sum(e, axis=-1, keepdims=True)


def kernel(x, W, b):
    b2 = b.reshape(1, NUM_EXPERTS)
    grid = (N_TOKENS // BT,)
    out_shape = (
        jax.ShapeDtypeStruct((N_TOKENS, NUM_EXPERTS), jnp.float32),
        jax.ShapeDtypeStruct((N_TOKENS, NUM_EXPERTS), jnp.float32),
    )
    logits, probs = pl.pallas_call(
        _router_body,
        grid=grid,
        in_specs=[
            pl.BlockSpec((BT, D_MODEL), lambda i: (i, 0)),
            pl.BlockSpec((NUM_EXPERTS, D_MODEL), lambda i: (0, 0)),
            pl.BlockSpec((1, NUM_EXPERTS), lambda i: (0, 0)),
        ],
        out_specs=(
            pl.BlockSpec((BT, NUM_EXPERTS), lambda i: (i, 0)),
            pl.BlockSpec((BT, NUM_EXPERTS), lambda i: (i, 0)),
        ),
        out_shape=out_shape,
        compiler_params=pltpu.CompilerParams(
            dimension_semantics=("parallel",),
        ),
    )(x, W, b2)
    return (logits, probs)
